# baseline (speedup 1.0000x reference)
import functools

import jax
import jax.numpy as jnp
from jax import lax
from jax.experimental import pallas as pl
from jax.experimental.pallas import tpu as pltpu
from jax.experimental.pallas import tpu_sc as plsc

E = 8
K = 2
D = 768
DFF = 768
OUT = 768
T = 4096
CAP = 1280
COEF = 0.01

TB = 512
NB = T // TB
MB = 256


def _gate_body(x_ref, wg_ref,
               s0_ref, s1_ref, v0_ref, v1_ref, g0_ref, g1_ref, aux_ref,
               imp_ref, carry_ref):
    pid = pl.program_id(0)

    @pl.when(pid == 0)
    def _init():
        imp_ref[...] = jnp.zeros((1, E), jnp.float32)
        carry_ref[...] = jnp.zeros((1, E), jnp.float32)

    x = x_ref[...]
    wg = wg_ref[...]
    logits = jnp.dot(x, wg, preferred_element_type=jnp.float32)

    iota = jax.lax.broadcasted_iota(jnp.int32, (TB, E), 1)
    m0 = jnp.max(logits, axis=1, keepdims=True)
    i0 = jnp.min(jnp.where(logits == m0, iota, E), axis=1, keepdims=True)
    masked = jnp.where(iota == i0, -jnp.inf, logits)
    m1 = jnp.max(masked, axis=1, keepdims=True)
    i1 = jnp.min(jnp.where(masked == m1, iota, E), axis=1, keepdims=True)

    g0 = 1.0 / (1.0 + jnp.exp(m1 - m0))
    g1 = 1.0 / (1.0 + jnp.exp(m0 - m1))

    ohA = (iota == i0).astype(jnp.float32)
    ohB = (iota == i1).astype(jnp.float32)

    imp_ref[...] += jnp.sum(ohA * g0 + ohB * g1, axis=0, keepdims=True)

    r = jax.lax.broadcasted_iota(jnp.int32, (TB, TB), 0)
    c = jax.lax.broadcasted_iota(jnp.int32, (TB, TB), 1)
    lt = (c < r).astype(jnp.float32)
    ab = ohA + ohB
    S = jnp.dot(lt, ab, preferred_element_type=jnp.float32) + carry_ref[...]
    pA = jnp.sum(S * ohA, axis=1, keepdims=True)
    pB = jnp.sum((S + ohA) * ohB, axis=1, keepdims=True)
    carry_ref[...] += jnp.sum(ab, axis=0, keepdims=True)

    kA = pA < CAP
    kB = pB < CAP
    s0_ref[...] = i0 * CAP + jnp.where(kA, pA.astype(jnp.int32), 0)
    s1_ref[...] = i1 * CAP + jnp.where(kB, pB.astype(jnp.int32), 0)
    tok = pid * TB + jax.lax.broadcasted_iota(jnp.int32, (TB, 1), 0)
    v0_ref[...] = jnp.where(kA, tok, -1)
    v1_ref[...] = jnp.where(kB, tok, -1)
    g0_ref[...] = jnp.where(kA, g0, 0.0)
    g1_ref[...] = jnp.where(kB, g1, 0.0)

    @pl.when(pid == NB - 1)
    def _fin():
        imp = imp_ref[...]
        mean = jnp.sum(imp) / E
        var = jnp.sum((imp - mean) ** 2) / E
        aux_ref[...] = jnp.full((1, 1), COEF * var / (mean * mean + 1e-10),
                                jnp.float32)


def _gating(x, Wg):
    out_shapes = (
        jax.ShapeDtypeStruct((T, 1), jnp.int32),
        jax.ShapeDtypeStruct((T, 1), jnp.int32),
        jax.ShapeDtypeStruct((T, 1), jnp.int32),
        jax.ShapeDtypeStruct((T, 1), jnp.int32),
        jax.ShapeDtypeStruct((T, 1), jnp.float32),
        jax.ShapeDtypeStruct((T, 1), jnp.float32),
        jax.ShapeDtypeStruct((1, 1), jnp.float32),
    )
    col = pl.BlockSpec((TB, 1), lambda i: (i, 0))
    return pl.pallas_call(
        _gate_body,
        grid=(NB,),
        in_specs=[
            pl.BlockSpec((TB, D), lambda i: (i, 0)),
            pl.BlockSpec((D, E), lambda i: (0, 0)),
        ],
        out_specs=(col, col, col, col, col, col,
                   pl.BlockSpec((1, 1), lambda i: (0, 0))),
        out_shape=out_shapes,
        scratch_shapes=[
            pltpu.VMEM((1, E), jnp.float32),
            pltpu.VMEM((1, E), jnp.float32),
        ],
    )(x, Wg)


def _ffn_body(ein_ref, w1_ref, b1_ref, w2_ref, b2_ref, out_ref):
    a = ein_ref[...].astype(jnp.bfloat16)
    h = jnp.maximum(
        jnp.dot(a, w1_ref[0].astype(jnp.bfloat16),
                preferred_element_type=jnp.float32) + b1_ref[0],
        0.0)
    out_ref[...] = (jnp.dot(h.astype(jnp.bfloat16),
                            w2_ref[0].astype(jnp.bfloat16),
                            preferred_element_type=jnp.float32)
                    + b2_ref[0])


def _ffn(ein, W1, b1, W2, b2):
    nm = CAP // MB
    return pl.pallas_call(
        _ffn_body,
        grid=(E, nm),
        in_specs=[
            pl.BlockSpec((MB, D), lambda e, m: (e * nm + m, 0)),
            pl.BlockSpec((1, D, DFF), lambda e, m: (e, 0, 0)),
            pl.BlockSpec((1, 1, DFF), lambda e, m: (e, 0, 0)),
            pl.BlockSpec((1, DFF, OUT), lambda e, m: (e, 0, 0)),
            pl.BlockSpec((1, 1, OUT), lambda e, m: (e, 0, 0)),
        ],
        out_specs=pl.BlockSpec((MB, OUT), lambda e, m: (e * nm + m, 0)),
        out_shape=jax.ShapeDtypeStruct((E * CAP, OUT), jnp.float32),
    )(ein, W1, b1, W2, b2)


_SC_MESH = plsc.VectorSubcoreMesh(core_axis_name="c", subcore_axis_name="s")
_NW = 32
_SLOTS = E * CAP
_SPW = _SLOTS // _NW
_GCH = 64
_TPW = T // _NW
_CCH = 64
_NV = D // 16


def _dispatch_body(x_hbm, s0_hbm, s1_hbm, v0_hbm, v1_hbm, ein_hbm,
                   idx0_v, idx1_v, s_v, v_v, rows_v, sem, sem2):
    wid = lax.axis_index("s") * 2 + lax.axis_index("c")
    tbase = wid * _TPW

    row_load = pltpu.async_copy(x_hbm.at[pl.ds(tbase, _TPW)], rows_v, sem)

    pltpu.sync_copy(s0_hbm.at[pl.ds(tbase, _TPW)], s_v)
    pltpu.sync_copy(v0_hbm.at[pl.ds(tbase, _TPW)], v_v)
    for i in range(_TPW // 16):
        sl = pl.ds(i * 16, 16)
        idx0_v[sl] = jnp.where(v_v[sl] >= 0, s_v[sl],
                               jnp.full((16,), _SLOTS, jnp.int32))
    pltpu.sync_copy(s1_hbm.at[pl.ds(tbase, _TPW)], s_v)
    pltpu.sync_copy(v1_hbm.at[pl.ds(tbase, _TPW)], v_v)
    for i in range(_TPW // 16):
        sl = pl.ds(i * 16, 16)
        idx1_v[sl] = jnp.where(v_v[sl] >= 0, s_v[sl],
                               jnp.full((16,), _SLOTS, jnp.int32))

    row_load.wait()
    c0 = pltpu.async_copy(rows_v, ein_hbm.at[idx0_v], sem)
    c1 = pltpu.async_copy(rows_v, ein_hbm.at[idx1_v], sem2)
    c0.wait()
    c1.wait()


@functools.partial(
    pl.kernel,
    out_type=jax.ShapeDtypeStruct((_SLOTS + 8, D), jnp.float32),
    mesh=_SC_MESH,
    scratch_types=[
        pltpu.VMEM((_TPW,), jnp.int32),
        pltpu.VMEM((_TPW,), jnp.int32),
        pltpu.VMEM((_TPW,), jnp.int32),
        pltpu.VMEM((_TPW,), jnp.int32),
        pltpu.VMEM((_TPW, D), jnp.float32),
        pltpu.SemaphoreType.DMA,
        pltpu.SemaphoreType.DMA,
    ],
    compiler_params=pltpu.CompilerParams(needs_layout_passes=False),
)
def _dispatch(x_hbm, s0_hbm, s1_hbm, v0_hbm, v1_hbm, ein_hbm,
              idx0_v, idx1_v, s_v, v_v, rows_v, sem, sem2):
    _dispatch_body(x_hbm, s0_hbm, s1_hbm, v0_hbm, v1_hbm, ein_hbm,
                   idx0_v, idx1_v, s_v, v_v, rows_v, sem, sem2)


def _combine_body(eo_hbm, s0_hbm, s1_hbm, g0_hbm, g1_hbm, out_hbm,
                  s0_v, s1_v, g0_v, g1_v, buf_v, acc_v, sem):
    wid = lax.axis_index("s") * 2 + lax.axis_index("c")
    tbase = wid * _TPW
    pltpu.sync_copy(s0_hbm.at[pl.ds(tbase, _TPW)], s0_v)
    pltpu.sync_copy(s1_hbm.at[pl.ds(tbase, _TPW)], s1_v)
    pltpu.sync_copy(g0_hbm.at[pl.ds(tbase, _TPW)], g0_v)
    pltpu.sync_copy(g1_hbm.at[pl.ds(tbase, _TPW)], g1_v)

    def _chunk(c, _):
        off = c * _CCH
        pltpu.async_copy(eo_hbm.at[s0_v.at[pl.ds(off, _CCH)]],
                         buf_v, sem).wait()

        def _mul(j, _):
            g = plsc.load_gather(g0_v, [jnp.full((16,), off + j, jnp.int32)])
            for v in range(_NV):
                sl = pl.ds(v * 16, 16)
                acc_v[j, sl] = buf_v[j, sl] * g
            return 0
        lax.fori_loop(0, _CCH, _mul, 0)

        pltpu.async_copy(eo_hbm.at[s1_v.at[pl.ds(off, _CCH)]],
                         buf_v, sem).wait()

        def _fma(j, _):
            g = plsc.load_gather(g1_v, [jnp.full((16,), off + j, jnp.int32)])
            for v in range(_NV):
                sl = pl.ds(v * 16, 16)
                acc_v[j, sl] = acc_v[j, sl] + buf_v[j, sl] * g
            return 0
        lax.fori_loop(0, _CCH, _fma, 0)

        pltpu.sync_copy(acc_v, out_hbm.at[pl.ds(tbase + off, _CCH)])
        return 0
    lax.fori_loop(0, _TPW // _CCH, _chunk, 0)


@functools.partial(
    pl.kernel,
    out_type=jax.ShapeDtypeStruct((T, OUT), jnp.float32),
    mesh=_SC_MESH,
    scratch_types=[
        pltpu.VMEM((_TPW,), jnp.int32),
        pltpu.VMEM((_TPW,), jnp.int32),
        pltpu.VMEM((_TPW,), jnp.float32),
        pltpu.VMEM((_TPW,), jnp.float32),
        pltpu.VMEM((_CCH, OUT), jnp.float32),
        pltpu.VMEM((_CCH, OUT), jnp.float32),
        pltpu.SemaphoreType.DMA,
    ],
    compiler_params=pltpu.CompilerParams(needs_layout_passes=False),
)
def _combine(eo_hbm, s0_hbm, s1_hbm, g0_hbm, g1_hbm, out_hbm,
             s0_v, s1_v, g0_v, g1_v, buf_v, acc_v, sem):
    _combine_body(eo_hbm, s0_hbm, s1_hbm, g0_hbm, g1_hbm, out_hbm,
                  s0_v, s1_v, g0_v, g1_v, buf_v, acc_v, sem)


def kernel(x, Wg, W1, b1, W2, b2):
    s0, s1, v0, v1, g0, g1, aux = _gating(x, Wg)
    s0 = s0.reshape(T)
    s1 = s1.reshape(T)
    v0 = v0.reshape(T)
    v1 = v1.reshape(T)
    g0 = g0.reshape(T)
    g1 = g1.reshape(T)

    ein = _dispatch(x, s0, s1, v0, v1)
    eo = _ffn(ein, W1, b1.reshape(E, 1, DFF), W2, b2.reshape(E, 1, OUT))
    out = _combine(eo, s0, s1, g0, g1)
    return out, aux.reshape(())


# speedup vs baseline: 1.1260x; 1.1260x over previous
import functools

import jax
import jax.numpy as jnp
from jax import lax
from jax.experimental import pallas as pl
from jax.experimental.pallas import tpu as pltpu
from jax.experimental.pallas import tpu_sc as plsc

E = 8
K = 2
D = 768
DFF = 768
OUT = 768
T = 4096
CAP = 1280
COEF = 0.01

TB = 512
NB = T // TB
MB = 256


def _pack_halves(a):
    h = a.shape[1] // 2
    lo = jax.lax.bitcast_convert_type(
        a[:, :h].astype(jnp.bfloat16), jnp.uint16).astype(jnp.int32)
    hi = jax.lax.bitcast_convert_type(
        a[:, h:].astype(jnp.bfloat16), jnp.uint16).astype(jnp.int32)
    return lo | (hi << 16)


def _unpack_halves(w):
    lo = jax.lax.bitcast_convert_type(
        (w & 0xFFFF).astype(jnp.uint16), jnp.bfloat16)
    hi = jax.lax.bitcast_convert_type(
        jax.lax.shift_right_logical(w, 16).astype(jnp.uint16), jnp.bfloat16)
    return jnp.concatenate([lo, hi], axis=1)


def _gate_body(x_ref, wg_ref,
               s0_ref, s1_ref, v0_ref, v1_ref, g0_ref, g1_ref, xpk_ref,
               aux_ref, imp_ref, carry_ref):
    pid = pl.program_id(0)

    @pl.when(pid == 0)
    def _init():
        imp_ref[...] = jnp.zeros((1, E), jnp.float32)
        carry_ref[...] = jnp.zeros((1, E), jnp.float32)

    x = x_ref[...]
    wg = wg_ref[...]
    logits = jnp.dot(x, wg, preferred_element_type=jnp.float32)
    xpk_ref[...] = _pack_halves(x)

    iota = jax.lax.broadcasted_iota(jnp.int32, (TB, E), 1)
    m0 = jnp.max(logits, axis=1, keepdims=True)
    i0 = jnp.min(jnp.where(logits == m0, iota, E), axis=1, keepdims=True)
    masked = jnp.where(iota == i0, -jnp.inf, logits)
    m1 = jnp.max(masked, axis=1, keepdims=True)
    i1 = jnp.min(jnp.where(masked == m1, iota, E), axis=1, keepdims=True)

    g0 = 1.0 / (1.0 + jnp.exp(m1 - m0))
    g1 = 1.0 / (1.0 + jnp.exp(m0 - m1))

    ohA = (iota == i0).astype(jnp.float32)
    ohB = (iota == i1).astype(jnp.float32)

    imp_ref[...] += jnp.sum(ohA * g0 + ohB * g1, axis=0, keepdims=True)

    r = jax.lax.broadcasted_iota(jnp.int32, (TB, TB), 0)
    c = jax.lax.broadcasted_iota(jnp.int32, (TB, TB), 1)
    lt = (c < r).astype(jnp.float32)
    ab = ohA + ohB
    S = jnp.dot(lt, ab, preferred_element_type=jnp.float32) + carry_ref[...]
    pA = jnp.sum(S * ohA, axis=1, keepdims=True)
    pB = jnp.sum((S + ohA) * ohB, axis=1, keepdims=True)
    carry_ref[...] += jnp.sum(ab, axis=0, keepdims=True)

    kA = pA < CAP
    kB = pB < CAP
    br = TB // 128
    s0_ref[...] = jnp.reshape(
        i0 * CAP + jnp.where(kA, pA.astype(jnp.int32), 0), (1, br, 128))
    s1_ref[...] = jnp.reshape(
        i1 * CAP + jnp.where(kB, pB.astype(jnp.int32), 0), (1, br, 128))
    tok = pid * TB + jax.lax.broadcasted_iota(jnp.int32, (TB, 1), 0)
    v0_ref[...] = jnp.reshape(jnp.where(kA, tok, -1), (1, br, 128))
    v1_ref[...] = jnp.reshape(jnp.where(kB, tok, -1), (1, br, 128))
    g0_ref[...] = jnp.reshape(jnp.where(kA, g0, 0.0), (1, br, 128))
    g1_ref[...] = jnp.reshape(jnp.where(kB, g1, 0.0), (1, br, 128))

    @pl.when(pid == NB - 1)
    def _fin():
        imp = imp_ref[...]
        mean = jnp.sum(imp) / E
        var = jnp.sum((imp - mean) ** 2) / E
        aux_ref[...] = jnp.full((1, 1), COEF * var / (mean * mean + 1e-10),
                                jnp.float32)


def _gating(x, Wg):
    br = TB // 128
    out_shapes = (
        jax.ShapeDtypeStruct((NB, br, 128), jnp.int32),
        jax.ShapeDtypeStruct((NB, br, 128), jnp.int32),
        jax.ShapeDtypeStruct((NB, br, 128), jnp.int32),
        jax.ShapeDtypeStruct((NB, br, 128), jnp.int32),
        jax.ShapeDtypeStruct((NB, br, 128), jnp.float32),
        jax.ShapeDtypeStruct((NB, br, 128), jnp.float32),
        jax.ShapeDtypeStruct((T, D // 2), jnp.int32),
        jax.ShapeDtypeStruct((1, 1), jnp.float32),
    )
    col = pl.BlockSpec((1, br, 128), lambda i: (i, 0, 0))
    return pl.pallas_call(
        _gate_body,
        grid=(NB,),
        in_specs=[
            pl.BlockSpec((TB, D), lambda i: (i, 0)),
            pl.BlockSpec((D, E), lambda i: (0, 0)),
        ],
        out_specs=(col, col, col, col, col, col,
                   pl.BlockSpec((TB, D // 2), lambda i: (i, 0)),
                   pl.BlockSpec((1, 1), lambda i: (0, 0))),
        out_shape=out_shapes,
        scratch_shapes=[
            pltpu.VMEM((1, E), jnp.float32),
            pltpu.VMEM((1, E), jnp.float32),
        ],
    )(x, Wg)


def _ffn_body(ein_ref, w1_ref, b1_ref, w2_ref, b2_ref, out_ref,
              w1s_ref, w2s_ref):
    @pl.when(pl.program_id(1) == 0)
    def _cvt():
        w1s_ref[...] = w1_ref[0].astype(jnp.bfloat16)
        w2s_ref[...] = w2_ref[0].astype(jnp.bfloat16)

    a = _unpack_halves(ein_ref[...])
    h = jnp.maximum(
        jnp.dot(a, w1s_ref[...], preferred_element_type=jnp.float32)
        + b1_ref[0], 0.0)
    o = (jnp.dot(h.astype(jnp.bfloat16), w2s_ref[...],
                 preferred_element_type=jnp.float32) + b2_ref[0])
    out_ref[...] = _pack_halves(o)


def _ffn(ein, W1, b1, W2, b2):
    nm = CAP // MB
    return pl.pallas_call(
        _ffn_body,
        grid=(E, nm),
        in_specs=[
            pl.BlockSpec((MB, D // 2), lambda e, m: (e * nm + m, 0)),
            pl.BlockSpec((1, D, DFF), lambda e, m: (e, 0, 0)),
            pl.BlockSpec((1, 1, DFF), lambda e, m: (e, 0, 0)),
            pl.BlockSpec((1, DFF, OUT), lambda e, m: (e, 0, 0)),
            pl.BlockSpec((1, 1, OUT), lambda e, m: (e, 0, 0)),
        ],
        out_specs=pl.BlockSpec((MB, OUT // 2), lambda e, m: (e * nm + m, 0)),
        out_shape=jax.ShapeDtypeStruct((E * CAP, OUT // 2), jnp.int32),
        scratch_shapes=[
            pltpu.VMEM((D, DFF), jnp.bfloat16),
            pltpu.VMEM((DFF, OUT), jnp.bfloat16),
        ],
    )(ein, W1, b1, W2, b2)


_SC_MESH = plsc.VectorSubcoreMesh(core_axis_name="c", subcore_axis_name="s")
_NW = 32
_SLOTS = E * CAP
_SPW = _SLOTS // _NW
_GCH = 64
_TPW = T // _NW
_CCH = 32
_NV = D // 16


def _dispatch_body(x_hbm, s0_hbm, s1_hbm, v0_hbm, v1_hbm, ein_hbm,
                   idx0_v, idx1_v, s_v, v_v, rows_v, sem, sem2):
    wid = lax.axis_index("s") * 2 + lax.axis_index("c")
    tbase = wid * _TPW

    row_load = pltpu.async_copy(x_hbm.at[pl.ds(tbase, _TPW)], rows_v, sem)

    pltpu.sync_copy(s0_hbm.at[pl.ds(tbase, _TPW)], s_v)
    pltpu.sync_copy(v0_hbm.at[pl.ds(tbase, _TPW)], v_v)
    for i in range(_TPW // 16):
        sl = pl.ds(i * 16, 16)
        idx0_v[sl] = jnp.where(v_v[sl] >= 0, s_v[sl],
                               jnp.full((16,), _SLOTS, jnp.int32))
    pltpu.sync_copy(s1_hbm.at[pl.ds(tbase, _TPW)], s_v)
    pltpu.sync_copy(v1_hbm.at[pl.ds(tbase, _TPW)], v_v)
    for i in range(_TPW // 16):
        sl = pl.ds(i * 16, 16)
        idx1_v[sl] = jnp.where(v_v[sl] >= 0, s_v[sl],
                               jnp.full((16,), _SLOTS, jnp.int32))

    row_load.wait()
    c0 = pltpu.async_copy(rows_v, ein_hbm.at[idx0_v], sem)
    c1 = pltpu.async_copy(rows_v, ein_hbm.at[idx1_v], sem2)
    c0.wait()
    c1.wait()


@functools.partial(
    pl.kernel,
    out_type=jax.ShapeDtypeStruct((_SLOTS + 8, D // 2), jnp.int32),
    mesh=_SC_MESH,
    scratch_types=[
        pltpu.VMEM((_TPW,), jnp.int32),
        pltpu.VMEM((_TPW,), jnp.int32),
        pltpu.VMEM((_TPW,), jnp.int32),
        pltpu.VMEM((_TPW,), jnp.int32),
        pltpu.VMEM((_TPW, D // 2), jnp.int32),
        pltpu.SemaphoreType.DMA,
        pltpu.SemaphoreType.DMA,
    ],
    compiler_params=pltpu.CompilerParams(needs_layout_passes=False),
)
def _dispatch(x_hbm, s0_hbm, s1_hbm, v0_hbm, v1_hbm, ein_hbm,
              idx0_v, idx1_v, s_v, v_v, rows_v, sem, sem2):
    _dispatch_body(x_hbm, s0_hbm, s1_hbm, v0_hbm, v1_hbm, ein_hbm,
                   idx0_v, idx1_v, s_v, v_v, rows_v, sem, sem2)


def _combine_body(eo_hbm, s0_hbm, s1_hbm, g0_hbm, g1_hbm, out_hbm,
                  s0_v, s1_v, g0_v, g1_v, bufA, bufB, acc, semA, semB, semO):
    wid = lax.axis_index("s") * 2 + lax.axis_index("c")
    tbase = wid * _TPW
    pltpu.sync_copy(s0_hbm.at[pl.ds(tbase, _TPW)], s0_v)
    pltpu.sync_copy(s1_hbm.at[pl.ds(tbase, _TPW)], s1_v)
    pltpu.sync_copy(g0_hbm.at[pl.ds(tbase, _TPW)], g0_v)
    pltpu.sync_copy(g1_hbm.at[pl.ds(tbase, _TPW)], g1_v)

    nch = _TPW // _CCH

    def _issue(c):
        p = c % 2
        a = pltpu.async_copy(eo_hbm.at[s0_v.at[pl.ds(c * _CCH, _CCH)]],
                             bufA[p], semA[p])
        b = pltpu.async_copy(eo_hbm.at[s1_v.at[pl.ds(c * _CCH, _CCH)]],
                             bufB[p], semB[p])
        return a, b

    pend = {0: _issue(0)}
    outd = [None, None]
    for c in range(nch):
        p = c % 2
        if c + 1 < nch:
            q = (c + 1) % 2
            if outd[q] is not None:
                outd[q].wait()
                outd[q] = None
            pend[c + 1] = _issue(c + 1)
        a, b = pend.pop(c)
        a.wait()
        b.wait()

        def _row(j, _, _p=p, _c=c):
            ga = plsc.load_gather(
                g0_v, [jnp.full((16,), _c * _CCH + j, jnp.int32)])
            gb = plsc.load_gather(
                g1_v, [jnp.full((16,), _c * _CCH + j, jnp.int32)])
            for v in range(_NV // 2):
                sl = pl.ds(v * 16, 16)
                sh = pl.ds(OUT // 2 + v * 16, 16)
                wa = bufA[_p][j, sl]
                wb = bufB[_p][j, sl]
                a_lo = plsc.bitcast(wa << 16, jnp.float32)
                a_hi = plsc.bitcast(wa & -65536, jnp.float32)
                b_lo = plsc.bitcast(wb << 16, jnp.float32)
                b_hi = plsc.bitcast(wb & -65536, jnp.float32)
                acc[_p][j, sl] = a_lo * ga + b_lo * gb
                acc[_p][j, sh] = a_hi * ga + b_hi * gb
            return 0
        lax.fori_loop(0, _CCH, _row, 0)

        outd[p] = pltpu.async_copy(
            acc[p], out_hbm.at[pl.ds(tbase + c * _CCH, _CCH)], semO[p])
    for p in range(2):
        if outd[p] is not None:
            outd[p].wait()


@functools.partial(
    pl.kernel,
    out_type=jax.ShapeDtypeStruct((T, OUT), jnp.float32),
    mesh=_SC_MESH,
    scratch_types=[
        pltpu.VMEM((_TPW,), jnp.int32),
        pltpu.VMEM((_TPW,), jnp.int32),
        pltpu.VMEM((_TPW,), jnp.float32),
        pltpu.VMEM((_TPW,), jnp.float32),
        [pltpu.VMEM((_CCH, OUT // 2), jnp.int32)] * 2,
        [pltpu.VMEM((_CCH, OUT // 2), jnp.int32)] * 2,
        [pltpu.VMEM((_CCH, OUT), jnp.float32)] * 2,
        [pltpu.SemaphoreType.DMA] * 2,
        [pltpu.SemaphoreType.DMA] * 2,
        [pltpu.SemaphoreType.DMA] * 2,
    ],
    compiler_params=pltpu.CompilerParams(needs_layout_passes=False),
)
def _combine(eo_hbm, s0_hbm, s1_hbm, g0_hbm, g1_hbm, out_hbm,
             s0_v, s1_v, g0_v, g1_v, bufA, bufB, acc, semA, semB, semO):
    _combine_body(eo_hbm, s0_hbm, s1_hbm, g0_hbm, g1_hbm, out_hbm,
                  s0_v, s1_v, g0_v, g1_v, bufA, bufB, acc, semA, semB, semO)


def kernel(x, Wg, W1, b1, W2, b2):
    s0, s1, v0, v1, g0, g1, xpk, aux = _gating(x, Wg)
    s0 = s0.reshape(T)
    s1 = s1.reshape(T)
    v0 = v0.reshape(T)
    v1 = v1.reshape(T)
    g0 = g0.reshape(T)
    g1 = g1.reshape(T)

    ein = _dispatch(xpk, s0, s1, v0, v1)
    eo = _ffn(ein, W1, b1.reshape(E, 1, DFF), W2, b2.reshape(E, 1, OUT))
    out = _combine(eo, s0, s1, g0, g1)
    return out, aux.reshape(())


# speedup vs baseline: 1.1429x; 1.0150x over previous
import functools

import jax
import jax.numpy as jnp
from jax import lax
from jax.experimental import pallas as pl
from jax.experimental.pallas import tpu as pltpu
from jax.experimental.pallas import tpu_sc as plsc

E = 8
K = 2
D = 768
DFF = 768
OUT = 768
T = 4096
CAP = 1280
COEF = 0.01

TB = 512
NB = T // TB
MB = 256


def _pack_halves(a):
    h = a.shape[1] // 2
    lo = jax.lax.bitcast_convert_type(
        a[:, :h].astype(jnp.bfloat16), jnp.uint16).astype(jnp.int32)
    hi = jax.lax.bitcast_convert_type(
        a[:, h:].astype(jnp.bfloat16), jnp.uint16).astype(jnp.int32)
    return lo | (hi << 16)


def _unpack_halves(w):
    lo = jax.lax.bitcast_convert_type(
        (w & 0xFFFF).astype(jnp.uint16), jnp.bfloat16)
    hi = jax.lax.bitcast_convert_type(
        jax.lax.shift_right_logical(w, 16).astype(jnp.uint16), jnp.bfloat16)
    return jnp.concatenate([lo, hi], axis=1)


def _gate_body(x_ref, wg_ref,
               s0_ref, s1_ref, v0_ref, v1_ref, g0_ref, g1_ref, xpk_ref,
               aux_ref, imp_ref, carry_ref):
    pid = pl.program_id(0)

    @pl.when(pid == 0)
    def _init():
        imp_ref[...] = jnp.zeros((1, E), jnp.float32)
        carry_ref[...] = jnp.zeros((1, E), jnp.float32)

    x = x_ref[...]
    wg = wg_ref[...]
    logits = jnp.dot(x, wg, preferred_element_type=jnp.float32)
    xpk_ref[...] = _pack_halves(x)

    iota = jax.lax.broadcasted_iota(jnp.int32, (TB, E), 1)
    m0 = jnp.max(logits, axis=1, keepdims=True)
    i0 = jnp.min(jnp.where(logits == m0, iota, E), axis=1, keepdims=True)
    masked = jnp.where(iota == i0, -jnp.inf, logits)
    m1 = jnp.max(masked, axis=1, keepdims=True)
    i1 = jnp.min(jnp.where(masked == m1, iota, E), axis=1, keepdims=True)

    g0 = 1.0 / (1.0 + jnp.exp(m1 - m0))
    g1 = 1.0 / (1.0 + jnp.exp(m0 - m1))

    ohA = (iota == i0).astype(jnp.float32)
    ohB = (iota == i1).astype(jnp.float32)

    imp_ref[...] += jnp.sum(ohA * g0 + ohB * g1, axis=0, keepdims=True)

    r = jax.lax.broadcasted_iota(jnp.int32, (TB, TB), 0)
    c = jax.lax.broadcasted_iota(jnp.int32, (TB, TB), 1)
    lt = (c < r).astype(jnp.float32)
    ab = ohA + ohB
    S = jnp.dot(lt, ab, preferred_element_type=jnp.float32) + carry_ref[...]
    pA = jnp.sum(S * ohA, axis=1, keepdims=True)
    pB = jnp.sum((S + ohA) * ohB, axis=1, keepdims=True)
    carry_ref[...] += jnp.sum(ab, axis=0, keepdims=True)

    kA = pA < CAP
    kB = pB < CAP
    br = TB // 128
    s0_ref[...] = jnp.reshape(
        i0 * CAP + jnp.where(kA, pA.astype(jnp.int32), 0), (1, br, 128))
    s1_ref[...] = jnp.reshape(
        i1 * CAP + jnp.where(kB, pB.astype(jnp.int32), 0), (1, br, 128))
    tok = pid * TB + jax.lax.broadcasted_iota(jnp.int32, (TB, 1), 0)
    v0_ref[...] = jnp.reshape(jnp.where(kA, tok, -1), (1, br, 128))
    v1_ref[...] = jnp.reshape(jnp.where(kB, tok, -1), (1, br, 128))
    g0_ref[...] = jnp.reshape(jnp.where(kA, g0, 0.0), (1, br, 128))
    g1_ref[...] = jnp.reshape(jnp.where(kB, g1, 0.0), (1, br, 128))

    @pl.when(pid == NB - 1)
    def _fin():
        imp = imp_ref[...]
        mean = jnp.sum(imp) / E
        var = jnp.sum((imp - mean) ** 2) / E
        aux_ref[...] = jnp.full((1, 1), COEF * var / (mean * mean + 1e-10),
                                jnp.float32)


def _gating(x, Wg):
    br = TB // 128
    out_shapes = (
        jax.ShapeDtypeStruct((NB, br, 128), jnp.int32),
        jax.ShapeDtypeStruct((NB, br, 128), jnp.int32),
        jax.ShapeDtypeStruct((NB, br, 128), jnp.int32),
        jax.ShapeDtypeStruct((NB, br, 128), jnp.int32),
        jax.ShapeDtypeStruct((NB, br, 128), jnp.float32),
        jax.ShapeDtypeStruct((NB, br, 128), jnp.float32),
        jax.ShapeDtypeStruct((T, D // 2), jnp.int32),
        jax.ShapeDtypeStruct((1, 1), jnp.float32),
    )
    col = pl.BlockSpec((1, br, 128), lambda i: (i, 0, 0))
    return pl.pallas_call(
        _gate_body,
        grid=(NB,),
        in_specs=[
            pl.BlockSpec((TB, D), lambda i: (i, 0)),
            pl.BlockSpec((D, E), lambda i: (0, 0)),
        ],
        out_specs=(col, col, col, col, col, col,
                   pl.BlockSpec((TB, D // 2), lambda i: (i, 0)),
                   pl.BlockSpec((1, 1), lambda i: (0, 0))),
        out_shape=out_shapes,
        scratch_shapes=[
            pltpu.VMEM((1, E), jnp.float32),
            pltpu.VMEM((1, E), jnp.float32),
        ],
    )(x, Wg)


def _ffn_body(ein_ref, w1_ref, b1_ref, w2_ref, b2_ref, out_ref,
              w1s_ref, w2s_ref):
    @pl.when(pl.program_id(1) == 0)
    def _cvt():
        w1s_ref[...] = w1_ref[0].astype(jnp.bfloat16)
        w2s_ref[...] = w2_ref[0].astype(jnp.bfloat16)

    a = _unpack_halves(ein_ref[...])
    h = jnp.maximum(
        jnp.dot(a, w1s_ref[...], preferred_element_type=jnp.float32)
        + b1_ref[0], 0.0)
    o = (jnp.dot(h.astype(jnp.bfloat16), w2s_ref[...],
                 preferred_element_type=jnp.float32) + b2_ref[0])
    out_ref[...] = _pack_halves(o)


def _ffn(ein, W1, b1, W2, b2):
    nm = CAP // MB
    return pl.pallas_call(
        _ffn_body,
        grid=(E, nm),
        in_specs=[
            pl.BlockSpec((MB, D // 2), lambda e, m: (e * nm + m, 0)),
            pl.BlockSpec((1, D, DFF), lambda e, m: (e, 0, 0)),
            pl.BlockSpec((1, 1, DFF), lambda e, m: (e, 0, 0)),
            pl.BlockSpec((1, DFF, OUT), lambda e, m: (e, 0, 0)),
            pl.BlockSpec((1, 1, OUT), lambda e, m: (e, 0, 0)),
        ],
        out_specs=pl.BlockSpec((MB, OUT // 2), lambda e, m: (e * nm + m, 0)),
        out_shape=jax.ShapeDtypeStruct((E * CAP, OUT // 2), jnp.int32),
        scratch_shapes=[
            pltpu.VMEM((D, DFF), jnp.bfloat16),
            pltpu.VMEM((DFF, OUT), jnp.bfloat16),
        ],
    )(ein, W1, b1, W2, b2)


_SC_MESH = plsc.VectorSubcoreMesh(core_axis_name="c", subcore_axis_name="s")
_NW = 32
_SLOTS = E * CAP
_SPW = _SLOTS // _NW
_GCH = 64
_TPW = T // _NW
_CCH = 32
_NV = D // 16


def _dispatch_body(x_hbm, s0_hbm, s1_hbm, v0_hbm, v1_hbm, ein_hbm,
                   idx0_v, idx1_v, s_v, v_v, rows_v, sem, sem2):
    wid = lax.axis_index("s") * 2 + lax.axis_index("c")
    tbase = wid * _TPW

    row_load = pltpu.async_copy(x_hbm.at[pl.ds(tbase, _TPW)], rows_v, sem)

    pltpu.sync_copy(s0_hbm.at[pl.ds(tbase, _TPW)], s_v)
    pltpu.sync_copy(v0_hbm.at[pl.ds(tbase, _TPW)], v_v)
    for i in range(_TPW // 16):
        sl = pl.ds(i * 16, 16)
        idx0_v[sl] = jnp.where(v_v[sl] >= 0, s_v[sl],
                               jnp.full((16,), _SLOTS, jnp.int32))
    pltpu.sync_copy(s1_hbm.at[pl.ds(tbase, _TPW)], s_v)
    pltpu.sync_copy(v1_hbm.at[pl.ds(tbase, _TPW)], v_v)
    for i in range(_TPW // 16):
        sl = pl.ds(i * 16, 16)
        idx1_v[sl] = jnp.where(v_v[sl] >= 0, s_v[sl],
                               jnp.full((16,), _SLOTS, jnp.int32))

    row_load.wait()
    c0 = pltpu.async_copy(rows_v, ein_hbm.at[idx0_v], sem)
    c1 = pltpu.async_copy(rows_v, ein_hbm.at[idx1_v], sem2)
    c0.wait()
    c1.wait()


@functools.partial(
    pl.kernel,
    out_type=jax.ShapeDtypeStruct((_SLOTS + 8, D // 2), jnp.int32),
    mesh=_SC_MESH,
    scratch_types=[
        pltpu.VMEM((_TPW,), jnp.int32),
        pltpu.VMEM((_TPW,), jnp.int32),
        pltpu.VMEM((_TPW,), jnp.int32),
        pltpu.VMEM((_TPW,), jnp.int32),
        pltpu.VMEM((_TPW, D // 2), jnp.int32),
        pltpu.SemaphoreType.DMA,
        pltpu.SemaphoreType.DMA,
    ],
    compiler_params=pltpu.CompilerParams(needs_layout_passes=False),
)
def _dispatch(x_hbm, s0_hbm, s1_hbm, v0_hbm, v1_hbm, ein_hbm,
              idx0_v, idx1_v, s_v, v_v, rows_v, sem, sem2):
    _dispatch_body(x_hbm, s0_hbm, s1_hbm, v0_hbm, v1_hbm, ein_hbm,
                   idx0_v, idx1_v, s_v, v_v, rows_v, sem, sem2)


def _combine_body(eo_hbm, s0_hbm, s1_hbm, g0_hbm, g1_hbm, out_hbm,
                  s0_v, s1_v, g0_v, g1_v, bufA, bufB, acc, semA, semB, semO):
    wid = lax.axis_index("s") * 2 + lax.axis_index("c")
    tbase = wid * _TPW
    pltpu.sync_copy(s0_hbm.at[pl.ds(tbase, _TPW)], s0_v)
    pltpu.sync_copy(s1_hbm.at[pl.ds(tbase, _TPW)], s1_v)
    pltpu.sync_copy(g0_hbm.at[pl.ds(tbase, _TPW)], g0_v)
    pltpu.sync_copy(g1_hbm.at[pl.ds(tbase, _TPW)], g1_v)

    nch = _TPW // _CCH

    def _issue(c):
        p = c % 2
        a = pltpu.async_copy(eo_hbm.at[s0_v.at[pl.ds(c * _CCH, _CCH)]],
                             bufA[p], semA[p])
        b = pltpu.async_copy(eo_hbm.at[s1_v.at[pl.ds(c * _CCH, _CCH)]],
                             bufB[p], semB[p])
        return a, b

    pend = {0: _issue(0)}
    outd = [None, None]
    for c in range(nch):
        p = c % 2
        if c + 1 < nch:
            q = (c + 1) % 2
            if outd[q] is not None:
                outd[q].wait()
                outd[q] = None
            pend[c + 1] = _issue(c + 1)
        a, b = pend.pop(c)
        a.wait()
        b.wait()

        def _row(j, _, _p=p, _c=c):
            ga = plsc.load_gather(
                g0_v, [jnp.full((16,), _c * _CCH + j, jnp.int32)])
            gb = plsc.load_gather(
                g1_v, [jnp.full((16,), _c * _CCH + j, jnp.int32)])
            gab = plsc.pack(ga, ga, format=plsc.PackFormat.INTERLEAVED)
            gbb = plsc.pack(gb, gb, format=plsc.PackFormat.INTERLEAVED)
            for v in range(_NV // 2):
                sl = pl.ds(v * 16, 16)
                sh = pl.ds(OUT // 2 + v * 16, 16)
                wa = plsc.bitcast(bufA[_p][j, sl], jnp.bfloat16)
                wb = plsc.bitcast(bufB[_p][j, sl], jnp.bfloat16)
                w = plsc.bitcast(wa * gab + wb * gbb, jnp.int32)
                acc[_p][j, sl] = plsc.bitcast(w << 16, jnp.float32)
                acc[_p][j, sh] = plsc.bitcast(w & -65536, jnp.float32)
            return 0
        lax.fori_loop(0, _CCH, _row, 0)

        outd[p] = pltpu.async_copy(
            acc[p], out_hbm.at[pl.ds(tbase + c * _CCH, _CCH)], semO[p])
    for p in range(2):
        if outd[p] is not None:
            outd[p].wait()


@functools.partial(
    pl.kernel,
    out_type=jax.ShapeDtypeStruct((T, OUT), jnp.float32),
    mesh=_SC_MESH,
    scratch_types=[
        pltpu.VMEM((_TPW,), jnp.int32),
        pltpu.VMEM((_TPW,), jnp.int32),
        pltpu.VMEM((_TPW,), jnp.float32),
        pltpu.VMEM((_TPW,), jnp.float32),
        [pltpu.VMEM((_CCH, OUT // 2), jnp.int32)] * 2,
        [pltpu.VMEM((_CCH, OUT // 2), jnp.int32)] * 2,
        [pltpu.VMEM((_CCH, OUT), jnp.float32)] * 2,
        [pltpu.SemaphoreType.DMA] * 2,
        [pltpu.SemaphoreType.DMA] * 2,
        [pltpu.SemaphoreType.DMA] * 2,
    ],
    compiler_params=pltpu.CompilerParams(needs_layout_passes=False),
)
def _combine(eo_hbm, s0_hbm, s1_hbm, g0_hbm, g1_hbm, out_hbm,
             s0_v, s1_v, g0_v, g1_v, bufA, bufB, acc, semA, semB, semO):
    _combine_body(eo_hbm, s0_hbm, s1_hbm, g0_hbm, g1_hbm, out_hbm,
                  s0_v, s1_v, g0_v, g1_v, bufA, bufB, acc, semA, semB, semO)


def kernel(x, Wg, W1, b1, W2, b2):
    s0, s1, v0, v1, g0, g1, xpk, aux = _gating(x, Wg)
    s0 = s0.reshape(T)
    s1 = s1.reshape(T)
    v0 = v0.reshape(T)
    v1 = v1.reshape(T)
    g0 = g0.reshape(T)
    g1 = g1.reshape(T)

    ein = _dispatch(xpk, s0, s1, v0, v1)
    eo = _ffn(ein, W1, b1.reshape(E, 1, DFF), W2, b2.reshape(E, 1, OUT))
    out = _combine(eo, s0, s1, g0, g1)
    return out, aux.reshape(())


# speedup vs baseline: 1.1579x; 1.0131x over previous
import functools

import jax
import jax.numpy as jnp
from jax import lax
from jax.experimental import pallas as pl
from jax.experimental.pallas import tpu as pltpu
from jax.experimental.pallas import tpu_sc as plsc

E = 8
K = 2
D = 768
DFF = 768
OUT = 768
T = 4096
CAP = 1280
COEF = 0.01

TB = 512
NB = T // TB
MB = 256


def _pack_halves(a):
    h = a.shape[1] // 2
    lo = jax.lax.bitcast_convert_type(
        a[:, :h].astype(jnp.bfloat16), jnp.uint16).astype(jnp.int32)
    hi = jax.lax.bitcast_convert_type(
        a[:, h:].astype(jnp.bfloat16), jnp.uint16).astype(jnp.int32)
    return lo | (hi << 16)


def _unpack_halves(w):
    lo = jax.lax.bitcast_convert_type(
        (w & 0xFFFF).astype(jnp.uint16), jnp.bfloat16)
    hi = jax.lax.bitcast_convert_type(
        jax.lax.shift_right_logical(w, 16).astype(jnp.uint16), jnp.bfloat16)
    return jnp.concatenate([lo, hi], axis=1)


def _gate_body(x_ref, wg_ref,
               s0_ref, s1_ref, v0_ref, v1_ref, g0_ref, g1_ref, xpk_ref,
               aux_ref, imp_ref, carry_ref):
    pid = pl.program_id(0)

    @pl.when(pid == 0)
    def _init():
        imp_ref[...] = jnp.zeros((1, E), jnp.float32)
        carry_ref[...] = jnp.zeros((1, E), jnp.float32)

    x = x_ref[...]
    wg = wg_ref[...]
    logits = jnp.dot(x, wg, preferred_element_type=jnp.float32)
    xpk_ref[...] = _pack_halves(x)

    iota = jax.lax.broadcasted_iota(jnp.int32, (TB, E), 1).astype(jnp.float32)
    m0 = jnp.max(logits, axis=1, keepdims=True)
    i0 = jnp.min(jnp.where(logits == m0, iota, float(E)),
                 axis=1, keepdims=True)
    masked = jnp.where(iota == i0, -jnp.inf, logits)
    m1 = jnp.max(masked, axis=1, keepdims=True)
    i1 = jnp.min(jnp.where(masked == m1, iota, float(E)),
                 axis=1, keepdims=True)

    g0 = 1.0 / (1.0 + jnp.exp(m1 - m0))
    g1 = 1.0 / (1.0 + jnp.exp(m0 - m1))

    ohA = (iota == i0).astype(jnp.float32)
    ohB = (iota == i1).astype(jnp.float32)

    ones_row = jnp.ones((1, TB), jnp.float32)
    imp_ref[...] += jnp.dot(ones_row, ohA * g0 + ohB * g1,
                            preferred_element_type=jnp.float32)

    r = jax.lax.broadcasted_iota(jnp.int32, (TB, TB), 0)
    c = jax.lax.broadcasted_iota(jnp.int32, (TB, TB), 1)
    lt = (c < r).astype(jnp.float32)
    ab = ohA + ohB
    S = jnp.dot(lt, ab, preferred_element_type=jnp.float32) + carry_ref[...]
    pA = jnp.sum(S * ohA, axis=1, keepdims=True)
    pB = jnp.sum((S + ohA) * ohB, axis=1, keepdims=True)
    carry_ref[...] += jnp.dot(ones_row, ab,
                              preferred_element_type=jnp.float32)

    kA = pA < CAP
    kB = pB < CAP
    br = TB // 128
    s0_ref[...] = jnp.reshape(
        (i0 * CAP + jnp.where(kA, pA, 0.0)).astype(jnp.int32), (1, br, 128))
    s1_ref[...] = jnp.reshape(
        (i1 * CAP + jnp.where(kB, pB, 0.0)).astype(jnp.int32), (1, br, 128))
    tok = pid * TB + jax.lax.broadcasted_iota(jnp.int32, (TB, 1), 0)
    v0_ref[...] = jnp.reshape(jnp.where(kA, tok, -1), (1, br, 128))
    v1_ref[...] = jnp.reshape(jnp.where(kB, tok, -1), (1, br, 128))
    g0_ref[...] = jnp.reshape(jnp.where(kA, g0, 0.0), (1, br, 128))
    g1_ref[...] = jnp.reshape(jnp.where(kB, g1, 0.0), (1, br, 128))

    @pl.when(pid == NB - 1)
    def _fin():
        imp = imp_ref[...]
        mean = jnp.sum(imp) / E
        var = jnp.sum((imp - mean) ** 2) / E
        aux_ref[...] = jnp.full((1, 1), COEF * var / (mean * mean + 1e-10),
                                jnp.float32)


def _gating(x, Wg):
    br = TB // 128
    out_shapes = (
        jax.ShapeDtypeStruct((NB, br, 128), jnp.int32),
        jax.ShapeDtypeStruct((NB, br, 128), jnp.int32),
        jax.ShapeDtypeStruct((NB, br, 128), jnp.int32),
        jax.ShapeDtypeStruct((NB, br, 128), jnp.int32),
        jax.ShapeDtypeStruct((NB, br, 128), jnp.float32),
        jax.ShapeDtypeStruct((NB, br, 128), jnp.float32),
        jax.ShapeDtypeStruct((T, D // 2), jnp.int32),
        jax.ShapeDtypeStruct((1, 1), jnp.float32),
    )
    col = pl.BlockSpec((1, br, 128), lambda i: (i, 0, 0))
    return pl.pallas_call(
        _gate_body,
        grid=(NB,),
        in_specs=[
            pl.BlockSpec((TB, D), lambda i: (i, 0)),
            pl.BlockSpec((D, E), lambda i: (0, 0)),
        ],
        out_specs=(col, col, col, col, col, col,
                   pl.BlockSpec((TB, D // 2), lambda i: (i, 0)),
                   pl.BlockSpec((1, 1), lambda i: (0, 0))),
        out_shape=out_shapes,
        scratch_shapes=[
            pltpu.VMEM((1, E), jnp.float32),
            pltpu.VMEM((1, E), jnp.float32),
        ],
    )(x, Wg)


def _ffn_body(ein_ref, w1_ref, b1_ref, w2_ref, b2_ref, out_ref,
              w1s_ref, w2s_ref):
    @pl.when(pl.program_id(1) == 0)
    def _cvt():
        w1s_ref[...] = w1_ref[0].astype(jnp.bfloat16)
        w2s_ref[...] = w2_ref[0].astype(jnp.bfloat16)

    a = _unpack_halves(ein_ref[...])
    h = jnp.maximum(
        jnp.dot(a, w1s_ref[...], preferred_element_type=jnp.float32)
        + b1_ref[0], 0.0)
    o = (jnp.dot(h.astype(jnp.bfloat16), w2s_ref[...],
                 preferred_element_type=jnp.float32) + b2_ref[0])
    out_ref[...] = _pack_halves(o)


def _ffn(ein, W1, b1, W2, b2):
    nm = CAP // MB
    return pl.pallas_call(
        _ffn_body,
        grid=(E, nm),
        in_specs=[
            pl.BlockSpec((MB, D // 2), lambda e, m: (e * nm + m, 0)),
            pl.BlockSpec((1, D, DFF), lambda e, m: (e, 0, 0)),
            pl.BlockSpec((1, 1, DFF), lambda e, m: (e, 0, 0)),
            pl.BlockSpec((1, DFF, OUT), lambda e, m: (e, 0, 0)),
            pl.BlockSpec((1, 1, OUT), lambda e, m: (e, 0, 0)),
        ],
        out_specs=pl.BlockSpec((MB, OUT // 2), lambda e, m: (e * nm + m, 0)),
        out_shape=jax.ShapeDtypeStruct((E * CAP, OUT // 2), jnp.int32),
        scratch_shapes=[
            pltpu.VMEM((D, DFF), jnp.bfloat16),
            pltpu.VMEM((DFF, OUT), jnp.bfloat16),
        ],
    )(ein, W1, b1, W2, b2)


_SC_MESH = plsc.VectorSubcoreMesh(core_axis_name="c", subcore_axis_name="s")
_NW = 32
_SLOTS = E * CAP
_SPW = _SLOTS // _NW
_GCH = 64
_TPW = T // _NW
_CCH = 16
_NV = D // 16


def _dispatch_body(x_hbm, s0_hbm, s1_hbm, v0_hbm, v1_hbm, ein_hbm,
                   idx0_v, idx1_v, s_v, v_v, rows_v, sem, sem2):
    wid = lax.axis_index("s") * 2 + lax.axis_index("c")
    tbase = wid * _TPW

    row_load = pltpu.async_copy(x_hbm.at[pl.ds(tbase, _TPW)], rows_v, sem)

    pltpu.sync_copy(s0_hbm.at[pl.ds(tbase, _TPW)], s_v)
    pltpu.sync_copy(v0_hbm.at[pl.ds(tbase, _TPW)], v_v)
    for i in range(_TPW // 16):
        sl = pl.ds(i * 16, 16)
        idx0_v[sl] = jnp.where(v_v[sl] >= 0, s_v[sl],
                               jnp.full((16,), _SLOTS, jnp.int32))
    pltpu.sync_copy(s1_hbm.at[pl.ds(tbase, _TPW)], s_v)
    pltpu.sync_copy(v1_hbm.at[pl.ds(tbase, _TPW)], v_v)
    for i in range(_TPW // 16):
        sl = pl.ds(i * 16, 16)
        idx1_v[sl] = jnp.where(v_v[sl] >= 0, s_v[sl],
                               jnp.full((16,), _SLOTS, jnp.int32))

    row_load.wait()
    c0 = pltpu.async_copy(rows_v, ein_hbm.at[idx0_v], sem)
    c1 = pltpu.async_copy(rows_v, ein_hbm.at[idx1_v], sem2)
    c0.wait()
    c1.wait()


@functools.partial(
    pl.kernel,
    out_type=jax.ShapeDtypeStruct((_SLOTS + 8, D // 2), jnp.int32),
    mesh=_SC_MESH,
    scratch_types=[
        pltpu.VMEM((_TPW,), jnp.int32),
        pltpu.VMEM((_TPW,), jnp.int32),
        pltpu.VMEM((_TPW,), jnp.int32),
        pltpu.VMEM((_TPW,), jnp.int32),
        pltpu.VMEM((_TPW, D // 2), jnp.int32),
        pltpu.SemaphoreType.DMA,
        pltpu.SemaphoreType.DMA,
    ],
    compiler_params=pltpu.CompilerParams(needs_layout_passes=False),
)
def _dispatch(x_hbm, s0_hbm, s1_hbm, v0_hbm, v1_hbm, ein_hbm,
              idx0_v, idx1_v, s_v, v_v, rows_v, sem, sem2):
    _dispatch_body(x_hbm, s0_hbm, s1_hbm, v0_hbm, v1_hbm, ein_hbm,
                   idx0_v, idx1_v, s_v, v_v, rows_v, sem, sem2)


def _combine_body(eo_hbm, s0_hbm, s1_hbm, g0_hbm, g1_hbm, out_hbm,
                  s0_v, s1_v, g0_v, g1_v, bufA, bufB, acc, semA, semB, semO):
    wid = lax.axis_index("s") * 2 + lax.axis_index("c")
    tbase = wid * _TPW
    pltpu.sync_copy(s0_hbm.at[pl.ds(tbase, _TPW)], s0_v)
    pltpu.sync_copy(s1_hbm.at[pl.ds(tbase, _TPW)], s1_v)
    pltpu.sync_copy(g0_hbm.at[pl.ds(tbase, _TPW)], g0_v)
    pltpu.sync_copy(g1_hbm.at[pl.ds(tbase, _TPW)], g1_v)

    nch = _TPW // _CCH

    pend = []
    for c in range(nch):
        a = pltpu.async_copy(eo_hbm.at[s0_v.at[pl.ds(c * _CCH, _CCH)]],
                             bufA[c], semA[c])
        b = pltpu.async_copy(eo_hbm.at[s1_v.at[pl.ds(c * _CCH, _CCH)]],
                             bufB[c], semB[c])
        pend.append((a, b))

    outd = [None, None]
    for c in range(nch):
        p = c % 2
        a, b = pend[c]
        a.wait()
        b.wait()
        if outd[p] is not None:
            outd[p].wait()
            outd[p] = None

        def _row(j, _, _b=c, _p=p, _c=c):
            ga = plsc.load_gather(
                g0_v, [jnp.full((16,), _c * _CCH + j, jnp.int32)])
            gb = plsc.load_gather(
                g1_v, [jnp.full((16,), _c * _CCH + j, jnp.int32)])
            gab = plsc.pack(ga, ga, format=plsc.PackFormat.INTERLEAVED)
            gbb = plsc.pack(gb, gb, format=plsc.PackFormat.INTERLEAVED)
            for v in range(_NV // 2):
                sl = pl.ds(v * 16, 16)
                sh = pl.ds(OUT // 2 + v * 16, 16)
                wa = plsc.bitcast(bufA[_b][j, sl], jnp.bfloat16)
                wb = plsc.bitcast(bufB[_b][j, sl], jnp.bfloat16)
                w = plsc.bitcast(wa * gab + wb * gbb, jnp.int32)
                acc[_p][j, sl] = plsc.bitcast(w << 16, jnp.float32)
                acc[_p][j, sh] = plsc.bitcast(w & -65536, jnp.float32)
            return 0
        lax.fori_loop(0, _CCH, _row, 0)

        outd[p] = pltpu.async_copy(
            acc[p], out_hbm.at[pl.ds(tbase + c * _CCH, _CCH)], semO[p])
    for p in range(2):
        if outd[p] is not None:
            outd[p].wait()


@functools.partial(
    pl.kernel,
    out_type=jax.ShapeDtypeStruct((T, OUT), jnp.float32),
    mesh=_SC_MESH,
    scratch_types=[
        pltpu.VMEM((_TPW,), jnp.int32),
        pltpu.VMEM((_TPW,), jnp.int32),
        pltpu.VMEM((_TPW,), jnp.float32),
        pltpu.VMEM((_TPW,), jnp.float32),
        [pltpu.VMEM((_CCH, OUT // 2), jnp.int32)] * (_TPW // _CCH),
        [pltpu.VMEM((_CCH, OUT // 2), jnp.int32)] * (_TPW // _CCH),
        [pltpu.VMEM((_CCH, OUT), jnp.float32)] * 2,
        [pltpu.SemaphoreType.DMA] * (_TPW // _CCH),
        [pltpu.SemaphoreType.DMA] * (_TPW // _CCH),
        [pltpu.SemaphoreType.DMA] * 2,
    ],
    compiler_params=pltpu.CompilerParams(needs_layout_passes=False),
)
def _combine(eo_hbm, s0_hbm, s1_hbm, g0_hbm, g1_hbm, out_hbm,
             s0_v, s1_v, g0_v, g1_v, bufA, bufB, acc, semA, semB, semO):
    _combine_body(eo_hbm, s0_hbm, s1_hbm, g0_hbm, g1_hbm, out_hbm,
                  s0_v, s1_v, g0_v, g1_v, bufA, bufB, acc, semA, semB, semO)


def kernel(x, Wg, W1, b1, W2, b2):
    s0, s1, v0, v1, g0, g1, xpk, aux = _gating(x, Wg)
    s0 = s0.reshape(T)
    s1 = s1.reshape(T)
    v0 = v0.reshape(T)
    v1 = v1.reshape(T)
    g0 = g0.reshape(T)
    g1 = g1.reshape(T)

    ein = _dispatch(xpk, s0, s1, v0, v1)
    eo = _ffn(ein, W1, b1.reshape(E, 1, DFF), W2, b2.reshape(E, 1, OUT))
    out = _combine(eo, s0, s1, g0, g1)
    return out, aux.reshape(())


# speedup vs baseline: 1.1605x; 1.0022x over previous
import functools

import jax
import jax.numpy as jnp
from jax import lax
from jax.experimental import pallas as pl
from jax.experimental.pallas import tpu as pltpu
from jax.experimental.pallas import tpu_sc as plsc

E = 8
K = 2
D = 768
DFF = 768
OUT = 768
T = 4096
CAP = 1280
COEF = 0.01

TB = 512
NB = T // TB
MB = 256


def _pack_halves(a):
    h = a.shape[1] // 2
    lo = jax.lax.bitcast_convert_type(
        a[:, :h].astype(jnp.bfloat16), jnp.uint16).astype(jnp.int32)
    hi = jax.lax.bitcast_convert_type(
        a[:, h:].astype(jnp.bfloat16), jnp.uint16).astype(jnp.int32)
    return lo | (hi << 16)


def _unpack_halves(w):
    lo = jax.lax.bitcast_convert_type(
        (w & 0xFFFF).astype(jnp.uint16), jnp.bfloat16)
    hi = jax.lax.bitcast_convert_type(
        jax.lax.shift_right_logical(w, 16).astype(jnp.uint16), jnp.bfloat16)
    return jnp.concatenate([lo, hi], axis=1)


def _gate_body(x_ref, wg_ref,
               s0_ref, s1_ref, v0_ref, v1_ref, g0_ref, g1_ref, xpk_ref,
               aux_ref, imp_ref, carry_ref):
    pid = pl.program_id(0)

    @pl.when(pid == 0)
    def _init():
        imp_ref[...] = jnp.zeros((1, E), jnp.float32)
        carry_ref[...] = jnp.zeros((1, E), jnp.float32)

    x = x_ref[...]
    wg = wg_ref[...]
    logits = jnp.dot(x, wg, preferred_element_type=jnp.float32)
    xpk_ref[...] = _pack_halves(x)

    iota = jax.lax.broadcasted_iota(jnp.int32, (TB, E), 1).astype(jnp.float32)
    m0 = jnp.max(logits, axis=1, keepdims=True)
    i0 = jnp.min(jnp.where(logits == m0, iota, float(E)),
                 axis=1, keepdims=True)
    masked = jnp.where(iota == i0, -jnp.inf, logits)
    m1 = jnp.max(masked, axis=1, keepdims=True)
    i1 = jnp.min(jnp.where(masked == m1, iota, float(E)),
                 axis=1, keepdims=True)

    g0 = 1.0 / (1.0 + jnp.exp(m1 - m0))
    g1 = 1.0 / (1.0 + jnp.exp(m0 - m1))

    ohA = (iota == i0).astype(jnp.float32)
    ohB = (iota == i1).astype(jnp.float32)

    ones_row = jnp.ones((1, TB), jnp.float32)
    imp_ref[...] += jnp.dot(ones_row, ohA * g0 + ohB * g1,
                            preferred_element_type=jnp.float32)

    r = jax.lax.broadcasted_iota(jnp.int32, (TB, TB), 0)
    c = jax.lax.broadcasted_iota(jnp.int32, (TB, TB), 1)
    lt = (c < r).astype(jnp.float32)
    ab = ohA + ohB
    S = jnp.dot(lt, ab, preferred_element_type=jnp.float32) + carry_ref[...]
    pA = jnp.sum(S * ohA, axis=1, keepdims=True)
    pB = jnp.sum((S + ohA) * ohB, axis=1, keepdims=True)
    carry_ref[...] += jnp.dot(ones_row, ab,
                              preferred_element_type=jnp.float32)

    kA = pA < CAP
    kB = pB < CAP
    br = TB // 128
    s0_ref[...] = jnp.reshape(
        (i0 * CAP + jnp.where(kA, pA, 0.0)).astype(jnp.int32), (1, br, 128))
    s1_ref[...] = jnp.reshape(
        (i1 * CAP + jnp.where(kB, pB, 0.0)).astype(jnp.int32), (1, br, 128))
    tok = pid * TB + jax.lax.broadcasted_iota(jnp.int32, (TB, 1), 0)
    v0_ref[...] = jnp.reshape(jnp.where(kA, tok, -1), (1, br, 128))
    v1_ref[...] = jnp.reshape(jnp.where(kB, tok, -1), (1, br, 128))
    g0_ref[...] = jnp.reshape(jnp.where(kA, g0, 0.0), (1, br, 128))
    g1_ref[...] = jnp.reshape(jnp.where(kB, g1, 0.0), (1, br, 128))

    @pl.when(pid == NB - 1)
    def _fin():
        imp = imp_ref[...]
        mean = jnp.sum(imp) / E
        var = jnp.sum((imp - mean) ** 2) / E
        aux_ref[...] = jnp.full((1, 1), COEF * var / (mean * mean + 1e-10),
                                jnp.float32)


def _gating(x, Wg):
    br = TB // 128
    out_shapes = (
        jax.ShapeDtypeStruct((NB, br, 128), jnp.int32),
        jax.ShapeDtypeStruct((NB, br, 128), jnp.int32),
        jax.ShapeDtypeStruct((NB, br, 128), jnp.int32),
        jax.ShapeDtypeStruct((NB, br, 128), jnp.int32),
        jax.ShapeDtypeStruct((NB, br, 128), jnp.float32),
        jax.ShapeDtypeStruct((NB, br, 128), jnp.float32),
        jax.ShapeDtypeStruct((T, D // 2), jnp.int32),
        jax.ShapeDtypeStruct((1, 1), jnp.float32),
    )
    col = pl.BlockSpec((1, br, 128), lambda i: (i, 0, 0))
    return pl.pallas_call(
        _gate_body,
        grid=(NB,),
        in_specs=[
            pl.BlockSpec((TB, D), lambda i: (i, 0)),
            pl.BlockSpec((D, E), lambda i: (0, 0)),
        ],
        out_specs=(col, col, col, col, col, col,
                   pl.BlockSpec((TB, D // 2), lambda i: (i, 0)),
                   pl.BlockSpec((1, 1), lambda i: (0, 0))),
        out_shape=out_shapes,
        scratch_shapes=[
            pltpu.VMEM((1, E), jnp.float32),
            pltpu.VMEM((1, E), jnp.float32),
        ],
    )(x, Wg)


def _ffn_body(ein_ref, w1_ref, b1_ref, w2_ref, b2_ref, out_ref,
              w1s_ref, w2s_ref):
    @pl.when(pl.program_id(1) == 0)
    def _cvt():
        w1s_ref[...] = w1_ref[0].astype(jnp.bfloat16)
        w2s_ref[...] = w2_ref[0].astype(jnp.bfloat16)

    a = _unpack_halves(ein_ref[...])
    h = jnp.maximum(
        jnp.dot(a, w1s_ref[...], preferred_element_type=jnp.float32)
        + b1_ref[0], 0.0)
    o = (jnp.dot(h.astype(jnp.bfloat16), w2s_ref[...],
                 preferred_element_type=jnp.float32) + b2_ref[0])
    out_ref[...] = _pack_halves(o)


def _ffn(ein, W1, b1, W2, b2):
    nm = CAP // MB
    return pl.pallas_call(
        _ffn_body,
        grid=(E, nm),
        in_specs=[
            pl.BlockSpec((MB, D // 2), lambda e, m: (e * nm + m, 0)),
            pl.BlockSpec((1, D, DFF), lambda e, m: (e, 0, 0)),
            pl.BlockSpec((1, 1, DFF), lambda e, m: (e, 0, 0)),
            pl.BlockSpec((1, DFF, OUT), lambda e, m: (e, 0, 0)),
            pl.BlockSpec((1, 1, OUT), lambda e, m: (e, 0, 0)),
        ],
        out_specs=pl.BlockSpec((MB, OUT // 2), lambda e, m: (e * nm + m, 0)),
        out_shape=jax.ShapeDtypeStruct((E * CAP, OUT // 2), jnp.int32),
        scratch_shapes=[
            pltpu.VMEM((D, DFF), jnp.bfloat16),
            pltpu.VMEM((DFF, OUT), jnp.bfloat16),
        ],
    )(ein, W1, b1, W2, b2)


_SC_MESH = plsc.VectorSubcoreMesh(core_axis_name="c", subcore_axis_name="s")
_NW = 32
_SLOTS = E * CAP
_TPW = T // _NW
_CCH = 16
_NV = D // 16


def _dispatch_body(x_hbm, s0_hbm, s1_hbm, v0_hbm, v1_hbm, ein_hbm,
                   idx0_v, idx1_v, s_v, v_v, rows_v, sem, sem2):
    wid = lax.axis_index("s") * 2 + lax.axis_index("c")
    tbase = wid * _TPW

    row_load = pltpu.async_copy(x_hbm.at[pl.ds(tbase, _TPW)], rows_v, sem)

    pltpu.sync_copy(s0_hbm.at[pl.ds(tbase, _TPW)], s_v)
    pltpu.sync_copy(v0_hbm.at[pl.ds(tbase, _TPW)], v_v)
    for i in range(_TPW // 16):
        sl = pl.ds(i * 16, 16)
        idx0_v[sl] = jnp.where(v_v[sl] >= 0, s_v[sl],
                               jnp.full((16,), _SLOTS, jnp.int32))
    pltpu.sync_copy(s1_hbm.at[pl.ds(tbase, _TPW)], s_v)
    pltpu.sync_copy(v1_hbm.at[pl.ds(tbase, _TPW)], v_v)
    for i in range(_TPW // 16):
        sl = pl.ds(i * 16, 16)
        idx1_v[sl] = jnp.where(v_v[sl] >= 0, s_v[sl],
                               jnp.full((16,), _SLOTS, jnp.int32))

    row_load.wait()
    c0 = pltpu.async_copy(rows_v, ein_hbm.at[idx0_v], sem)
    c1 = pltpu.async_copy(rows_v, ein_hbm.at[idx1_v], sem2)
    c0.wait()
    c1.wait()


@functools.partial(
    pl.kernel,
    out_type=jax.ShapeDtypeStruct((_SLOTS + 8, D // 2), jnp.int32),
    mesh=_SC_MESH,
    scratch_types=[
        pltpu.VMEM((_TPW,), jnp.int32),
        pltpu.VMEM((_TPW,), jnp.int32),
        pltpu.VMEM((_TPW,), jnp.int32),
        pltpu.VMEM((_TPW,), jnp.int32),
        pltpu.VMEM((_TPW, D // 2), jnp.int32),
        pltpu.SemaphoreType.DMA,
        pltpu.SemaphoreType.DMA,
    ],
    compiler_params=pltpu.CompilerParams(needs_layout_passes=False),
)
def _dispatch(x_hbm, s0_hbm, s1_hbm, v0_hbm, v1_hbm, ein_hbm,
              idx0_v, idx1_v, s_v, v_v, rows_v, sem, sem2):
    _dispatch_body(x_hbm, s0_hbm, s1_hbm, v0_hbm, v1_hbm, ein_hbm,
                   idx0_v, idx1_v, s_v, v_v, rows_v, sem, sem2)


def _combine_body(eo_hbm, s0_hbm, s1_hbm, g0_hbm, g1_hbm, out_hbm,
                  s0_v, s1_v, g0_v, g1_v, bufA, bufB, acc, semA, semB, semO):
    wid = lax.axis_index("s") * 2 + lax.axis_index("c")
    tbase = wid * _TPW
    pltpu.sync_copy(s0_hbm.at[pl.ds(tbase, _TPW)], s0_v)
    pltpu.sync_copy(s1_hbm.at[pl.ds(tbase, _TPW)], s1_v)
    pltpu.sync_copy(g0_hbm.at[pl.ds(tbase, _TPW)], g0_v)
    pltpu.sync_copy(g1_hbm.at[pl.ds(tbase, _TPW)], g1_v)

    nch = _TPW // _CCH

    pend = []
    for c in range(nch):
        a = pltpu.async_copy(eo_hbm.at[s0_v.at[pl.ds(c * _CCH, _CCH)]],
                             bufA[c], semA[c])
        b = pltpu.async_copy(eo_hbm.at[s1_v.at[pl.ds(c * _CCH, _CCH)]],
                             bufB[c], semB[c])
        pend.append((a, b))

    outd = [None, None]
    for c in range(nch):
        p = c % 2
        a, b = pend[c]
        a.wait()
        b.wait()
        if outd[p] is not None:
            outd[p].wait()
            outd[p] = None

        def _row(j, _, _b=c, _p=p, _c=c):
            ga = plsc.load_gather(
                g0_v, [jnp.full((16,), _c * _CCH + j, jnp.int32)])
            gb = plsc.load_gather(
                g1_v, [jnp.full((16,), _c * _CCH + j, jnp.int32)])
            gab = plsc.pack(ga, ga, format=plsc.PackFormat.INTERLEAVED)
            gbb = plsc.pack(gb, gb, format=plsc.PackFormat.INTERLEAVED)
            for v in range(_NV // 2):
                sl = pl.ds(v * 16, 16)
                sh = pl.ds(OUT // 2 + v * 16, 16)
                wa = plsc.bitcast(bufA[_b][j, sl], jnp.bfloat16)
                wb = plsc.bitcast(bufB[_b][j, sl], jnp.bfloat16)
                w = plsc.bitcast(wa * gab + wb * gbb, jnp.int32)
                acc[_p][j, sl] = plsc.bitcast(w << 16, jnp.float32)
                acc[_p][j, sh] = plsc.bitcast(w & -65536, jnp.float32)
            return 0
        lax.fori_loop(0, _CCH, _row, 0)

        outd[p] = pltpu.async_copy(
            acc[p], out_hbm.at[pl.ds(tbase + c * _CCH, _CCH)], semO[p])
    for p in range(2):
        if outd[p] is not None:
            outd[p].wait()


@functools.partial(
    pl.kernel,
    out_type=jax.ShapeDtypeStruct((T, OUT), jnp.float32),
    mesh=_SC_MESH,
    scratch_types=[
        pltpu.VMEM((_TPW,), jnp.int32),
        pltpu.VMEM((_TPW,), jnp.int32),
        pltpu.VMEM((_TPW,), jnp.float32),
        pltpu.VMEM((_TPW,), jnp.float32),
        [pltpu.VMEM((_CCH, OUT // 2), jnp.int32)] * (_TPW // _CCH),
        [pltpu.VMEM((_CCH, OUT // 2), jnp.int32)] * (_TPW // _CCH),
        [pltpu.VMEM((_CCH, OUT), jnp.float32)] * 2,
        [pltpu.SemaphoreType.DMA] * (_TPW // _CCH),
        [pltpu.SemaphoreType.DMA] * (_TPW // _CCH),
        [pltpu.SemaphoreType.DMA] * 2,
    ],
    compiler_params=pltpu.CompilerParams(needs_layout_passes=False),
)
def _combine(eo_hbm, s0_hbm, s1_hbm, g0_hbm, g1_hbm, out_hbm,
             s0_v, s1_v, g0_v, g1_v, bufA, bufB, acc, semA, semB, semO):
    _combine_body(eo_hbm, s0_hbm, s1_hbm, g0_hbm, g1_hbm, out_hbm,
                  s0_v, s1_v, g0_v, g1_v, bufA, bufB, acc, semA, semB, semO)


def kernel(x, Wg, W1, b1, W2, b2):
    s0, s1, v0, v1, g0, g1, xpk, aux = _gating(x, Wg)
    s0 = s0.reshape(T)
    s1 = s1.reshape(T)
    v0 = v0.reshape(T)
    v1 = v1.reshape(T)
    g0 = g0.reshape(T)
    g1 = g1.reshape(T)

    ein = _dispatch(xpk, s0, s1, v0, v1)
    eo = _ffn(ein, W1, b1.reshape(E, 1, DFF), W2, b2.reshape(E, 1, OUT))
    out = _combine(eo, s0, s1, g0, g1)
    return out, aux.reshape(())
